# baseline (speedup 1.0000x reference)
import functools

import numpy as np
import jax
import jax.numpy as jnp
from jax import lax
from jax.experimental import pallas as pl
from jax.experimental.pallas import tpu as pltpu
from jax.experimental.pallas import tpu_sc as plsc

_N_NODES = 10000
_N_PAD = 10240
_N_EDGES = 320000
_D = 128
_EPS = 1e-5

_SC_CHUNK = 1280
_N_CHUNKS = _N_EDGES // _SC_CHUNK
_GROUPS = _SC_CHUNK // 16


def _build_mats():
    n = _D
    j = np.arange(n)
    a_x = np.zeros((n, n))
    a_b = np.zeros((n, n))
    m2 = np.zeros((n, n))
    a_x[0] = 1.0
    a_b[0] = 1.0
    a_x[1] = (-1.0) ** j
    a_b[1] = (-1.0) ** j
    m2[0] = 1.0 / n
    m2[1] = ((-1.0) ** j) / n
    for b in range(1, 64):
        c = np.cos(2 * np.pi * b * j / n)
        s = np.sin(2 * np.pi * b * j / n)
        a_x[2 * b] = c
        a_x[2 * b + 1] = s
        a_b[2 * b] = c
        a_b[2 * b + 1] = -s
        m2[2 * b] = 2 * c / n
        m2[2 * b + 1] = -2 * s / n
    return (a_x.astype(np.float32), a_b.astype(np.float32),
            m2.astype(np.float32))


_AX, _AB, _M2 = _build_mats()
_SHIFT_IDX = (np.arange(_D)[:, None] + np.arange(_D)[None, :]) % _D


def _transform_body(a_ref, x_ref, o_ref):
    o_ref[...] = lax.dot_general(
        a_ref[...], x_ref[...], (((1,), (1,)), ((), ())),
        preferred_element_type=jnp.float32,
        precision=lax.Precision.HIGHEST)


def _transform(mat, arr, blk):
    rows = arr.shape[0]
    return pl.pallas_call(
        _transform_body,
        grid=(rows // blk,),
        in_specs=[pl.BlockSpec((_D, _D), lambda i: (0, 0)),
                  pl.BlockSpec((blk, _D), lambda i: (i, 0))],
        out_specs=pl.BlockSpec((_D, blk), lambda i: (0, i)),
        out_shape=jax.ShapeDtypeStruct((_D, rows), jnp.float32),
    )(mat, arr)


def _sc_edge_pass(fxt, fbt, src, dst, norm_pad):
    mesh = plsc.VectorSubcoreMesh(core_axis_name="c", subcore_axis_name="s")

    @functools.partial(
        pl.kernel,
        out_type=jax.ShapeDtypeStruct((_D, _N_PAD), jnp.float32),
        mesh=mesh,
        scratch_types=[
            pltpu.VMEM((4 * _N_PAD,), jnp.float32),
            pltpu.VMEM((4 * _N_PAD,), jnp.float32),
            pltpu.VMEM((_N_PAD,), jnp.float32),
            pltpu.VMEM((2, _SC_CHUNK), jnp.int32),
            pltpu.VMEM((2, _SC_CHUNK), jnp.int32),
            pltpu.VMEM((2, 4 * _SC_CHUNK), jnp.float32),
            pltpu.SemaphoreType.DMA,
            pltpu.SemaphoreType.DMA,
        ],
        compiler_params=pltpu.CompilerParams(needs_layout_passes=False),
    )
    def k(fxt_hbm, fbt_hbm, src_hbm, dst_hbm, norm_hbm, out_hbm,
          fx_v, agg_v, norm_v, src_v, dst_v, fb_v, sem0, sem1):
        sems = [sem0, sem1]
        wid = lax.axis_index("c") * 16 + lax.axis_index("s")
        r0 = wid * 4
        is0 = wid == 0
        s_a = jnp.where(is0, 0.0, 1.0).astype(jnp.float32)

        for r in range(4):
            pltpu.sync_copy(fxt_hbm.at[r0 + r],
                            fx_v.at[pl.ds(r * _N_PAD, _N_PAD)])
        pltpu.sync_copy(norm_hbm.at[:], norm_v)

        @plsc.parallel_loop(0, 4 * _N_PAD // 16, unroll=4)
        def zero_body(i):
            agg_v[pl.ds(i * 16, 16)] = jnp.zeros((16,), jnp.float32)

        def chunk_copies(g, b):
            off = g * _SC_CHUNK
            copies = [
                pltpu.make_async_copy(
                    src_hbm.at[pl.ds(off, _SC_CHUNK)], src_v.at[b], sems[b]),
                pltpu.make_async_copy(
                    dst_hbm.at[pl.ds(off, _SC_CHUNK)], dst_v.at[b], sems[b]),
            ]
            for r in range(4):
                copies.append(pltpu.make_async_copy(
                    fbt_hbm.at[r0 + r, pl.ds(off, _SC_CHUNK)],
                    fb_v.at[b, pl.ds(r * _SC_CHUNK, _SC_CHUNK)], sems[b]))
            return copies

        for c in chunk_copies(0, 0):
            c.start()
        for c in chunk_copies(1, 1):
            c.start()

        def do_chunk(g, b):
            for c in chunk_copies(g, b):
                c.wait()

            @plsc.parallel_loop(0, _GROUPS, unroll=8)
            def group_body(i):
                sl = pl.ds(i * 16, 16)
                s_idx = src_v[b, sl]
                d_idx = dst_v[b, sl]
                a0 = plsc.load_gather(fx_v, [s_idx])
                a1 = plsc.load_gather(fx_v, [s_idx + _N_PAD])
                a2 = plsc.load_gather(fx_v, [s_idx + 2 * _N_PAD])
                a3 = plsc.load_gather(fx_v, [s_idx + 3 * _N_PAD])
                b0 = fb_v[b, pl.ds(i * 16, 16)]
                b1 = fb_v[b, pl.ds(_SC_CHUNK + i * 16, 16)]
                b2 = fb_v[b, pl.ds(2 * _SC_CHUNK + i * 16, 16)]
                b3 = fb_v[b, pl.ds(3 * _SC_CHUNK + i * 16, 16)]
                t11 = a1 * b1
                cr_a = a0 * b0 - s_a * t11
                ci_a = s_a * (a0 * b1 + a1 * b0) + (1.0 - s_a) * t11
                cr_b = a2 * b2 - a3 * b3
                ci_b = a2 * b3 + a3 * b2
                plsc.addupdate_scatter(agg_v, [d_idx], cr_a)
                plsc.addupdate_scatter(agg_v, [d_idx + _N_PAD], ci_a)
                plsc.addupdate_scatter(agg_v, [d_idx + 2 * _N_PAD], cr_b)
                plsc.addupdate_scatter(agg_v, [d_idx + 3 * _N_PAD], ci_b)

            @pl.when(g + 2 < _N_CHUNKS)
            def _():
                for c in chunk_copies(g + 2, b):
                    c.start()

        def outer_body(h, carry):
            g = h * 2
            do_chunk(g, 0)
            do_chunk(g + 1, 1)
            return carry
        lax.fori_loop(0, _N_CHUNKS // 2, outer_body, 0)

        @plsc.parallel_loop(0, _N_PAD // 16, unroll=2)
        def norm_body(i):
            sl = pl.ds(i * 16, 16)
            nv = norm_v[sl]
            for r in range(4):
                rsl = pl.ds(r * _N_PAD + i * 16, 16)
                agg_v[rsl] = agg_v[rsl] * nv

        for r in range(4):
            pltpu.sync_copy(agg_v.at[pl.ds(r * _N_PAD, _N_PAD)],
                            out_hbm.at[r0 + r])

    return k(fxt, fbt, src, dst, norm_pad)


_NODE_BLK = 2048


def _nodes_body(aggt_ref, m3_ref, x_ref, wle_ref, bias_ref, pre_ref, stats_ref):
    i = pl.program_id(0)
    aggm = lax.dot_general(
        aggt_ref[...], m3_ref[...], (((0,), (0,)), ((), ())),
        preferred_element_type=jnp.float32, precision=lax.Precision.HIGHEST)
    loopm = jnp.dot(x_ref[...], wle_ref[...],
                    preferred_element_type=jnp.float32,
                    precision=lax.Precision.HIGHEST)
    pre = aggm + loopm + bias_ref[...]
    pre_ref[...] = pre
    rowid = lax.broadcasted_iota(jnp.int32, (_NODE_BLK, 1), 0) + i * _NODE_BLK
    prem = jnp.where(rowid < _N_NODES, pre, 0.0)

    @pl.when(i == 0)
    def _():
        stats_ref[...] = jnp.zeros_like(stats_ref)

    stats_ref[0:1, :] += jnp.sum(prem, axis=0, keepdims=True)
    stats_ref[1:2, :] += jnp.sum(prem * prem, axis=0, keepdims=True)


def _nodes_call(aggt, m3, x_pad, wle, bias):
    return pl.pallas_call(
        _nodes_body,
        grid=(_N_PAD // _NODE_BLK,),
        in_specs=[pl.BlockSpec((_D, _NODE_BLK), lambda i: (0, i)),
                  pl.BlockSpec((_D, _D), lambda i: (0, 0)),
                  pl.BlockSpec((_NODE_BLK, _D), lambda i: (i, 0)),
                  pl.BlockSpec((_D, _D), lambda i: (0, 0)),
                  pl.BlockSpec((1, _D), lambda i: (0, 0))],
        out_specs=[pl.BlockSpec((_NODE_BLK, _D), lambda i: (i, 0)),
                   pl.BlockSpec((2, _D), lambda i: (0, 0))],
        out_shape=[jax.ShapeDtypeStruct((_N_PAD, _D), jnp.float32),
                   jax.ShapeDtypeStruct((2, _D), jnp.float32)],
    )(aggt, m3, x_pad, wle, bias)


def _bn_body(pre_ref, stats_ref, gam_ref, bet_ref, out_ref):
    s = stats_ref[...]
    mean = s[0:1, :] * (1.0 / _N_NODES)
    var = s[1:2, :] * (1.0 / _N_NODES) - mean * mean
    inv = lax.rsqrt(var + _EPS)
    out_ref[...] = jnp.tanh(
        (pre_ref[...] - mean) * (inv * gam_ref[...]) + bet_ref[...])


def _bn_call(pre, stats, gam, bet):
    return pl.pallas_call(
        _bn_body,
        grid=(_N_PAD // _NODE_BLK,),
        in_specs=[pl.BlockSpec((_NODE_BLK, _D), lambda i: (i, 0)),
                  pl.BlockSpec((2, _D), lambda i: (0, 0)),
                  pl.BlockSpec((1, _D), lambda i: (0, 0)),
                  pl.BlockSpec((1, _D), lambda i: (0, 0))],
        out_specs=pl.BlockSpec((_NODE_BLK, _D), lambda i: (i, 0)),
        out_shape=jax.ShapeDtypeStruct((_N_PAD, _D), jnp.float32),
    )(pre, stats, gam, bet)


def kernel(x, edge_index, edge_attr, norm, w_loop, w_in, loop_rel, w_bias,
           bn_gamma, bn_beta):
    src = edge_index[0].astype(jnp.int32)
    dst = edge_index[1].astype(jnp.int32)

    ax = jnp.asarray(_AX)
    ab = jnp.asarray(_AB)
    m3 = jnp.dot(jnp.asarray(_M2), w_in, precision=lax.Precision.HIGHEST)
    shift = loop_rel[0][jnp.asarray(_SHIFT_IDX)]
    wle = jnp.dot(shift, w_loop, precision=lax.Precision.HIGHEST)

    x_pad = jnp.pad(x, ((0, _N_PAD - _N_NODES), (0, 0)))
    norm_pad = jnp.pad(norm[:, 0], (0, _N_PAD - _N_NODES))

    fxt = _transform(ax, x_pad, 2048)
    fbt = _transform(ab, edge_attr, 6400)

    aggt = _sc_edge_pass(fxt, fbt, src, dst, norm_pad)

    pre, stats = _nodes_call(aggt, m3, x_pad, wle, w_bias.reshape(1, _D))
    outp = _bn_call(pre, stats, bn_gamma.reshape(1, _D),
                    bn_beta.reshape(1, _D))
    return outp[:_N_NODES]


# speedup vs baseline: 1.1024x; 1.1024x over previous
import functools

import numpy as np
import jax
import jax.numpy as jnp
from jax import lax
from jax.experimental import pallas as pl
from jax.experimental.pallas import tpu as pltpu
from jax.experimental.pallas import tpu_sc as plsc

_N_NODES = 10000
_N_PAD = 10240
_N_EDGES = 320000
_D = 128
_EPS = 1e-5

_E_HALF = _N_EDGES // 2
_SC_CHUNK = 640
_N_CHUNKS = _E_HALF // _SC_CHUNK
_GROUPS = _SC_CHUNK // 16


def _build_mats():
    n = _D
    j = np.arange(n)
    a_x = np.zeros((n, n))
    a_b = np.zeros((n, n))
    m2 = np.zeros((n, n))
    a_x[0] = 1.0
    a_b[0] = 1.0
    a_x[1] = (-1.0) ** j
    a_b[1] = (-1.0) ** j
    m2[0] = 1.0 / n
    m2[1] = ((-1.0) ** j) / n
    for b in range(1, 64):
        c = np.cos(2 * np.pi * b * j / n)
        s = np.sin(2 * np.pi * b * j / n)
        a_x[2 * b] = c
        a_x[2 * b + 1] = s
        a_b[2 * b] = c
        a_b[2 * b + 1] = -s
        m2[2 * b] = 2 * c / n
        m2[2 * b + 1] = -2 * s / n
    return (a_x.astype(np.float32), a_b.astype(np.float32),
            m2.astype(np.float32))


_AX, _AB, _M2 = _build_mats()
_SHIFT_IDX = (np.arange(_D)[:, None] + np.arange(_D)[None, :]) % _D


def _transform_body(a_ref, x_ref, o_ref):
    o_ref[...] = lax.dot_general(
        a_ref[...], x_ref[...], (((1,), (1,)), ((), ())),
        preferred_element_type=jnp.float32,
        precision=lax.Precision.HIGHEST)


def _transform(mat, arr, blk):
    rows = arr.shape[0]
    return pl.pallas_call(
        _transform_body,
        grid=(rows // blk,),
        in_specs=[pl.BlockSpec((_D, _D), lambda i: (0, 0)),
                  pl.BlockSpec((blk, _D), lambda i: (i, 0))],
        out_specs=pl.BlockSpec((_D, blk), lambda i: (0, i)),
        out_shape=jax.ShapeDtypeStruct((_D, rows), jnp.float32),
    )(mat, arr)


_EBLK = 6400


def _packfb_body(a_ref, lo_ref, hi_ref, o_ref):
    vlo = lax.dot_general(
        a_ref[...], lo_ref[...], (((1,), (1,)), ((), ())),
        preferred_element_type=jnp.float32, precision=lax.Precision.HIGHEST)
    vhi = lax.dot_general(
        a_ref[...], hi_ref[...], (((1,), (1,)), ((), ())),
        preferred_element_type=jnp.float32, precision=lax.Precision.HIGHEST)
    lo32 = lax.convert_element_type(
        lax.bitcast_convert_type(vlo.astype(jnp.bfloat16), jnp.uint16),
        jnp.uint32)
    hi32 = lax.convert_element_type(
        lax.bitcast_convert_type(vhi.astype(jnp.bfloat16), jnp.uint16),
        jnp.uint32)
    o_ref[...] = lax.bitcast_convert_type((hi32 << 16) | lo32, jnp.int32)


def _transform_pack(mat, arr):
    nblk = _E_HALF // _EBLK
    return pl.pallas_call(
        _packfb_body,
        grid=(nblk,),
        in_specs=[pl.BlockSpec((_D, _D), lambda i: (0, 0)),
                  pl.BlockSpec((_EBLK, _D), lambda i: (i, 0)),
                  pl.BlockSpec((_EBLK, _D), lambda i: (i + nblk, 0))],
        out_specs=pl.BlockSpec((_D, _EBLK), lambda i: (0, i)),
        out_shape=jax.ShapeDtypeStruct((_D, _E_HALF), jnp.int32),
    )(mat, arr, arr)


def _sc_edge_pass(fxt, fbt, sd, norm_pad):
    mesh = plsc.VectorSubcoreMesh(core_axis_name="c", subcore_axis_name="s")

    @functools.partial(
        pl.kernel,
        out_type=jax.ShapeDtypeStruct((_D, _N_PAD), jnp.float32),
        mesh=mesh,
        scratch_types=[
            pltpu.VMEM((4 * _N_PAD,), jnp.float32),
            pltpu.VMEM((4 * _N_PAD,), jnp.float32),
            pltpu.VMEM((_N_PAD,), jnp.float32),
            pltpu.VMEM((2 * _SC_CHUNK,), jnp.int32),
            pltpu.VMEM((2 * _SC_CHUNK,), jnp.int32),
            pltpu.VMEM((8 * _SC_CHUNK,), jnp.int32),
            pltpu.SemaphoreType.DMA,
            pltpu.SemaphoreType.DMA,
        ],
        compiler_params=pltpu.CompilerParams(needs_layout_passes=False),
    )
    def k(fxt_hbm, fbt_hbm, sd_hbm, norm_hbm, out_hbm,
          fx_v, agg_v, norm_v, sdlo_v, sdhi_v, fb_v, sem0, sem1):
        sems = [sem0, sem1]
        wid = lax.axis_index("c") * 16 + lax.axis_index("s")
        r0 = wid * 4
        is0 = wid == 0
        s_a = jnp.where(is0, 0.0, 1.0).astype(jnp.float32)

        for r in range(4):
            pltpu.sync_copy(fxt_hbm.at[r0 + r],
                            fx_v.at[pl.ds(r * _N_PAD, _N_PAD)])
        pltpu.sync_copy(norm_hbm.at[:], norm_v)

        @plsc.parallel_loop(0, 4 * _N_PAD // 16, unroll=4)
        def zero_body(i):
            agg_v[pl.ds(i * 16, 16)] = jnp.zeros((16,), jnp.float32)

        def chunk_copies(g, b):
            off = g * _SC_CHUNK
            copies = [
                pltpu.make_async_copy(
                    sd_hbm.at[pl.ds(off, _SC_CHUNK)],
                    sdlo_v.at[pl.ds(b * _SC_CHUNK, _SC_CHUNK)], sems[b]),
                pltpu.make_async_copy(
                    sd_hbm.at[pl.ds(_E_HALF + off, _SC_CHUNK)],
                    sdhi_v.at[pl.ds(b * _SC_CHUNK, _SC_CHUNK)], sems[b]),
            ]
            for r in range(4):
                copies.append(pltpu.make_async_copy(
                    fbt_hbm.at[r0 + r, pl.ds(off, _SC_CHUNK)],
                    fb_v.at[pl.ds((b * 4 + r) * _SC_CHUNK, _SC_CHUNK)],
                    sems[b]))
            return copies

        for c in chunk_copies(0, 0):
            c.start()
        for c in chunk_copies(1, 1):
            c.start()

        def do_chunk(g, b):
            for c in chunk_copies(g, b):
                c.wait()

            @plsc.parallel_loop(0, _GROUPS, unroll=4)
            def group_body(i):
                sl16 = pl.ds(b * _SC_CHUNK + i * 16, 16)
                p_lo = sdlo_v[sl16]
                p_hi = sdhi_v[sl16]
                sd2 = ((p_lo & 0xFFFF, p_lo >> 16),
                       (p_hi & 0xFFFF, p_hi >> 16))
                fb = []
                for r in range(4):
                    w = fb_v[pl.ds((b * 4 + r) * _SC_CHUNK + i * 16, 16)]
                    fb.append(plsc.unpack(
                        plsc.bitcast(w, jnp.bfloat16),
                        format=plsc.PackFormat.INTERLEAVED,
                        preferred_element_type=jnp.float32))
                for h in range(2):
                    s_idx, d_idx = sd2[h]
                    a0 = plsc.load_gather(fx_v, [s_idx])
                    a1 = plsc.load_gather(fx_v, [s_idx + _N_PAD])
                    a2 = plsc.load_gather(fx_v, [s_idx + 2 * _N_PAD])
                    a3 = plsc.load_gather(fx_v, [s_idx + 3 * _N_PAD])
                    b0, b1, b2, b3 = fb[0][h], fb[1][h], fb[2][h], fb[3][h]
                    t11 = a1 * b1
                    cr_a = a0 * b0 - s_a * t11
                    ci_a = s_a * (a0 * b1 + a1 * b0) + (1.0 - s_a) * t11
                    cr_b = a2 * b2 - a3 * b3
                    ci_b = a2 * b3 + a3 * b2
                    plsc.addupdate_scatter(agg_v, [d_idx], cr_a)
                    plsc.addupdate_scatter(agg_v, [d_idx + _N_PAD], ci_a)
                    plsc.addupdate_scatter(agg_v, [d_idx + 2 * _N_PAD], cr_b)
                    plsc.addupdate_scatter(agg_v, [d_idx + 3 * _N_PAD], ci_b)

            @pl.when(g + 2 < _N_CHUNKS)
            def _():
                for c in chunk_copies(g + 2, b):
                    c.start()

        def outer_body(h, carry):
            g = h * 2
            do_chunk(g, 0)
            do_chunk(g + 1, 1)
            return carry
        lax.fori_loop(0, _N_CHUNKS // 2, outer_body, 0)

        @plsc.parallel_loop(0, _N_PAD // 16, unroll=2)
        def norm_body(i):
            sl = pl.ds(i * 16, 16)
            nv = norm_v[sl]
            for r in range(4):
                rsl = pl.ds(r * _N_PAD + i * 16, 16)
                agg_v[rsl] = agg_v[rsl] * nv

        for r in range(4):
            pltpu.sync_copy(agg_v.at[pl.ds(r * _N_PAD, _N_PAD)],
                            out_hbm.at[r0 + r])

    return k(fxt, fbt, sd, norm_pad)


_NODE_BLK = 2048


def _nodes_body(aggt_ref, m3_ref, x_ref, wle_ref, bias_ref, pre_ref, stats_ref):
    i = pl.program_id(0)
    aggm = lax.dot_general(
        aggt_ref[...], m3_ref[...], (((0,), (0,)), ((), ())),
        preferred_element_type=jnp.float32, precision=lax.Precision.HIGHEST)
    loopm = jnp.dot(x_ref[...], wle_ref[...],
                    preferred_element_type=jnp.float32,
                    precision=lax.Precision.HIGHEST)
    pre = aggm + loopm + bias_ref[...]
    pre_ref[...] = pre
    rowid = lax.broadcasted_iota(jnp.int32, (_NODE_BLK, 1), 0) + i * _NODE_BLK
    prem = jnp.where(rowid < _N_NODES, pre, 0.0)

    @pl.when(i == 0)
    def _():
        stats_ref[...] = jnp.zeros_like(stats_ref)

    stats_ref[0:1, :] += jnp.sum(prem, axis=0, keepdims=True)
    stats_ref[1:2, :] += jnp.sum(prem * prem, axis=0, keepdims=True)


def _nodes_call(aggt, m3, x_pad, wle, bias):
    return pl.pallas_call(
        _nodes_body,
        grid=(_N_PAD // _NODE_BLK,),
        in_specs=[pl.BlockSpec((_D, _NODE_BLK), lambda i: (0, i)),
                  pl.BlockSpec((_D, _D), lambda i: (0, 0)),
                  pl.BlockSpec((_NODE_BLK, _D), lambda i: (i, 0)),
                  pl.BlockSpec((_D, _D), lambda i: (0, 0)),
                  pl.BlockSpec((1, _D), lambda i: (0, 0))],
        out_specs=[pl.BlockSpec((_NODE_BLK, _D), lambda i: (i, 0)),
                   pl.BlockSpec((2, _D), lambda i: (0, 0))],
        out_shape=[jax.ShapeDtypeStruct((_N_PAD, _D), jnp.float32),
                   jax.ShapeDtypeStruct((2, _D), jnp.float32)],
    )(aggt, m3, x_pad, wle, bias)


def _bn_body(pre_ref, stats_ref, gam_ref, bet_ref, out_ref):
    s = stats_ref[...]
    mean = s[0:1, :] * (1.0 / _N_NODES)
    var = s[1:2, :] * (1.0 / _N_NODES) - mean * mean
    inv = lax.rsqrt(var + _EPS)
    out_ref[...] = jnp.tanh(
        (pre_ref[...] - mean) * (inv * gam_ref[...]) + bet_ref[...])


def _bn_call(pre, stats, gam, bet):
    return pl.pallas_call(
        _bn_body,
        grid=(_N_PAD // _NODE_BLK,),
        in_specs=[pl.BlockSpec((_NODE_BLK, _D), lambda i: (i, 0)),
                  pl.BlockSpec((2, _D), lambda i: (0, 0)),
                  pl.BlockSpec((1, _D), lambda i: (0, 0)),
                  pl.BlockSpec((1, _D), lambda i: (0, 0))],
        out_specs=pl.BlockSpec((_NODE_BLK, _D), lambda i: (i, 0)),
        out_shape=jax.ShapeDtypeStruct((_N_PAD, _D), jnp.float32),
    )(pre, stats, gam, bet)


def kernel(x, edge_index, edge_attr, norm, w_loop, w_in, loop_rel, w_bias,
           bn_gamma, bn_beta):
    src = edge_index[0].astype(jnp.int32)
    dst = edge_index[1].astype(jnp.int32)

    ax = jnp.asarray(_AX)
    ab = jnp.asarray(_AB)
    m3 = jnp.dot(jnp.asarray(_M2), w_in, precision=lax.Precision.HIGHEST)
    shift = loop_rel[0][jnp.asarray(_SHIFT_IDX)]
    wle = jnp.dot(shift, w_loop, precision=lax.Precision.HIGHEST)

    x_pad = jnp.pad(x, ((0, _N_PAD - _N_NODES), (0, 0)))
    norm_pad = jnp.pad(norm[:, 0], (0, _N_PAD - _N_NODES))

    fxt = _transform(ax, x_pad, 2048)
    fbt = _transform_pack(ab, edge_attr)

    sd = ((dst << 16) | src).astype(jnp.int32)
    aggt = _sc_edge_pass(fxt, fbt, sd, norm_pad)

    pre, stats = _nodes_call(aggt, m3, x_pad, wle, w_bias.reshape(1, _D))
    outp = _bn_call(pre, stats, bn_gamma.reshape(1, _D),
                    bn_beta.reshape(1, _D))
    return outp[:_N_NODES]


# speedup vs baseline: 1.2701x; 1.1521x over previous
import functools

import numpy as np
import jax
import jax.numpy as jnp
from jax import lax
from jax.experimental import pallas as pl
from jax.experimental.pallas import tpu as pltpu
from jax.experimental.pallas import tpu_sc as plsc

_N_NODES = 10000
_N_PAD = 10240
_N_EDGES = 320000
_D = 128
_EPS = 1e-5

_E_HALF = _N_EDGES // 2
_SC_CHUNK = 640
_N_CHUNKS = _E_HALF // _SC_CHUNK
_GROUPS = _SC_CHUNK // 16


def _build_mats():
    n = _D
    j = np.arange(n)
    a_x = np.zeros((n, n))
    a_b = np.zeros((n, n))
    m2 = np.zeros((n, n))
    a_x[0] = 1.0
    a_b[0] = 1.0
    a_x[1] = (-1.0) ** j
    a_b[1] = (-1.0) ** j
    m2[0] = 1.0 / n
    m2[1] = ((-1.0) ** j) / n
    for b in range(1, 64):
        c = np.cos(2 * np.pi * b * j / n)
        s = np.sin(2 * np.pi * b * j / n)
        a_x[2 * b] = c
        a_x[2 * b + 1] = s
        a_b[2 * b] = c
        a_b[2 * b + 1] = -s
        m2[2 * b] = 2 * c / n
        m2[2 * b + 1] = -2 * s / n
    return (a_x.astype(np.float32), a_b.astype(np.float32),
            m2.astype(np.float32))


_AX, _AB, _M2 = _build_mats()
_SHIFT_IDX = (np.arange(_D)[:, None] + np.arange(_D)[None, :]) % _D


def _transform_body(a_ref, x_ref, o_ref):
    o_ref[...] = lax.dot_general(
        a_ref[...], x_ref[...], (((1,), (1,)), ((), ())),
        preferred_element_type=jnp.float32,
        precision=lax.Precision.HIGHEST)


def _transform(mat, arr, blk):
    rows = arr.shape[0]
    return pl.pallas_call(
        _transform_body,
        grid=(rows // blk,),
        in_specs=[pl.BlockSpec((_D, _D), lambda i: (0, 0)),
                  pl.BlockSpec((blk, _D), lambda i: (i, 0))],
        out_specs=pl.BlockSpec((_D, blk), lambda i: (0, i)),
        out_shape=jax.ShapeDtypeStruct((_D, rows), jnp.float32),
    )(mat, arr)


_EBLK = 6400


def _packfb_body(a_ref, lo_ref, hi_ref, o_ref):
    vlo = lax.dot_general(
        a_ref[...], lo_ref[...], (((1,), (1,)), ((), ())),
        preferred_element_type=jnp.float32)
    vhi = lax.dot_general(
        a_ref[...], hi_ref[...], (((1,), (1,)), ((), ())),
        preferred_element_type=jnp.float32)
    lo32 = lax.convert_element_type(
        lax.bitcast_convert_type(vlo.astype(jnp.bfloat16), jnp.uint16),
        jnp.uint32)
    hi32 = lax.convert_element_type(
        lax.bitcast_convert_type(vhi.astype(jnp.bfloat16), jnp.uint16),
        jnp.uint32)
    o_ref[...] = lax.bitcast_convert_type((hi32 << 16) | lo32, jnp.int32)


def _transform_pack(mat, arr):
    nblk = _E_HALF // _EBLK
    return pl.pallas_call(
        _packfb_body,
        grid=(nblk,),
        in_specs=[pl.BlockSpec((_D, _D), lambda i: (0, 0)),
                  pl.BlockSpec((_EBLK, _D), lambda i: (i, 0)),
                  pl.BlockSpec((_EBLK, _D), lambda i: (i + nblk, 0))],
        out_specs=pl.BlockSpec((_D, _EBLK), lambda i: (0, i)),
        out_shape=jax.ShapeDtypeStruct((_D, _E_HALF), jnp.int32),
    )(mat, arr, arr)


def _sc_edge_pass(fxt, fbt, sd, norm_pad):
    mesh = plsc.VectorSubcoreMesh(core_axis_name="c", subcore_axis_name="s")

    @functools.partial(
        pl.kernel,
        out_type=jax.ShapeDtypeStruct((_D, _N_PAD), jnp.float32),
        mesh=mesh,
        scratch_types=[
            pltpu.VMEM((4 * _N_PAD,), jnp.float32),
            pltpu.VMEM((4 * _N_PAD,), jnp.float32),
            pltpu.VMEM((_N_PAD,), jnp.float32),
            pltpu.VMEM((2 * _SC_CHUNK,), jnp.int32),
            pltpu.VMEM((2 * _SC_CHUNK,), jnp.int32),
            pltpu.VMEM((8 * _SC_CHUNK,), jnp.int32),
            pltpu.SemaphoreType.DMA,
            pltpu.SemaphoreType.DMA,
        ],
        compiler_params=pltpu.CompilerParams(needs_layout_passes=False),
    )
    def k(fxt_hbm, fbt_hbm, sd_hbm, norm_hbm, out_hbm,
          fx_v, agg_v, norm_v, sdlo_v, sdhi_v, fb_v, sem0, sem1):
        sems = [sem0, sem1]
        wid = lax.axis_index("c") * 16 + lax.axis_index("s")
        r0 = wid * 4
        is0 = wid == 0
        s_a = jnp.where(is0, 0.0, 1.0).astype(jnp.float32)

        for r in range(4):
            pltpu.sync_copy(fxt_hbm.at[r0 + r],
                            fx_v.at[pl.ds(r * _N_PAD, _N_PAD)])
        pltpu.sync_copy(norm_hbm.at[:], norm_v)

        @plsc.parallel_loop(0, 4 * _N_PAD // 16, unroll=4)
        def zero_body(i):
            agg_v[pl.ds(i * 16, 16)] = jnp.zeros((16,), jnp.float32)

        def chunk_copies(g, b):
            off = g * _SC_CHUNK
            copies = [
                pltpu.make_async_copy(
                    sd_hbm.at[pl.ds(off, _SC_CHUNK)],
                    sdlo_v.at[pl.ds(b * _SC_CHUNK, _SC_CHUNK)], sems[b]),
                pltpu.make_async_copy(
                    sd_hbm.at[pl.ds(_E_HALF + off, _SC_CHUNK)],
                    sdhi_v.at[pl.ds(b * _SC_CHUNK, _SC_CHUNK)], sems[b]),
            ]
            for r in range(4):
                copies.append(pltpu.make_async_copy(
                    fbt_hbm.at[r0 + r, pl.ds(off, _SC_CHUNK)],
                    fb_v.at[pl.ds((b * 4 + r) * _SC_CHUNK, _SC_CHUNK)],
                    sems[b]))
            return copies

        for c in chunk_copies(0, 0):
            c.start()
        for c in chunk_copies(1, 1):
            c.start()

        def do_chunk(g, b):
            for c in chunk_copies(g, b):
                c.wait()

            @plsc.parallel_loop(0, _GROUPS, unroll=4)
            def group_body(i):
                sl16 = pl.ds(b * _SC_CHUNK + i * 16, 16)
                p_lo = sdlo_v[sl16]
                p_hi = sdhi_v[sl16]
                sd2 = ((p_lo & 0xFFFF, p_lo >> 16),
                       (p_hi & 0xFFFF, p_hi >> 16))
                fb = []
                for r in range(4):
                    w = fb_v[pl.ds((b * 4 + r) * _SC_CHUNK + i * 16, 16)]
                    fb.append(plsc.unpack(
                        plsc.bitcast(w, jnp.bfloat16),
                        format=plsc.PackFormat.INTERLEAVED,
                        preferred_element_type=jnp.float32))
                for h in range(2):
                    s_idx, d_idx = sd2[h]
                    a0 = plsc.load_gather(fx_v, [s_idx])
                    a1 = plsc.load_gather(fx_v, [s_idx + _N_PAD])
                    a2 = plsc.load_gather(fx_v, [s_idx + 2 * _N_PAD])
                    a3 = plsc.load_gather(fx_v, [s_idx + 3 * _N_PAD])
                    b0, b1, b2, b3 = fb[0][h], fb[1][h], fb[2][h], fb[3][h]
                    t11 = a1 * b1
                    cr_a = a0 * b0 - s_a * t11
                    ci_a = s_a * (a0 * b1 + a1 * b0) + (1.0 - s_a) * t11
                    cr_b = a2 * b2 - a3 * b3
                    ci_b = a2 * b3 + a3 * b2
                    plsc.addupdate_scatter(agg_v, [d_idx], cr_a)
                    plsc.addupdate_scatter(agg_v, [d_idx + _N_PAD], ci_a)
                    plsc.addupdate_scatter(agg_v, [d_idx + 2 * _N_PAD], cr_b)
                    plsc.addupdate_scatter(agg_v, [d_idx + 3 * _N_PAD], ci_b)

            @pl.when(g + 2 < _N_CHUNKS)
            def _():
                for c in chunk_copies(g + 2, b):
                    c.start()

        def outer_body(h, carry):
            g = h * 2
            do_chunk(g, 0)
            do_chunk(g + 1, 1)
            return carry
        lax.fori_loop(0, _N_CHUNKS // 2, outer_body, 0)

        @plsc.parallel_loop(0, _N_PAD // 16, unroll=2)
        def norm_body(i):
            sl = pl.ds(i * 16, 16)
            nv = norm_v[sl]
            for r in range(4):
                rsl = pl.ds(r * _N_PAD + i * 16, 16)
                agg_v[rsl] = agg_v[rsl] * nv

        for r in range(4):
            pltpu.sync_copy(agg_v.at[pl.ds(r * _N_PAD, _N_PAD)],
                            out_hbm.at[r0 + r])

    return k(fxt, fbt, sd, norm_pad)


_NODE_BLK = 2048


def _nodes_body(aggt_ref, m3_ref, x_ref, wle_ref, bias_ref, pre_ref, stats_ref):
    i = pl.program_id(0)
    aggm = lax.dot_general(
        aggt_ref[...], m3_ref[...], (((0,), (0,)), ((), ())),
        preferred_element_type=jnp.float32, precision=lax.Precision.HIGHEST)
    loopm = jnp.dot(x_ref[...], wle_ref[...],
                    preferred_element_type=jnp.float32,
                    precision=lax.Precision.HIGHEST)
    pre = aggm + loopm + bias_ref[...]
    pre_ref[...] = pre
    rowid = lax.broadcasted_iota(jnp.int32, (_NODE_BLK, 1), 0) + i * _NODE_BLK
    prem = jnp.where(rowid < _N_NODES, pre, 0.0)

    @pl.when(i == 0)
    def _():
        stats_ref[...] = jnp.zeros_like(stats_ref)

    stats_ref[0:1, :] += jnp.sum(prem, axis=0, keepdims=True)
    stats_ref[1:2, :] += jnp.sum(prem * prem, axis=0, keepdims=True)


def _nodes_call(aggt, m3, x_pad, wle, bias):
    return pl.pallas_call(
        _nodes_body,
        grid=(_N_PAD // _NODE_BLK,),
        in_specs=[pl.BlockSpec((_D, _NODE_BLK), lambda i: (0, i)),
                  pl.BlockSpec((_D, _D), lambda i: (0, 0)),
                  pl.BlockSpec((_NODE_BLK, _D), lambda i: (i, 0)),
                  pl.BlockSpec((_D, _D), lambda i: (0, 0)),
                  pl.BlockSpec((1, _D), lambda i: (0, 0))],
        out_specs=[pl.BlockSpec((_NODE_BLK, _D), lambda i: (i, 0)),
                   pl.BlockSpec((2, _D), lambda i: (0, 0))],
        out_shape=[jax.ShapeDtypeStruct((_N_PAD, _D), jnp.float32),
                   jax.ShapeDtypeStruct((2, _D), jnp.float32)],
    )(aggt, m3, x_pad, wle, bias)


def _bn_body(pre_ref, stats_ref, gam_ref, bet_ref, out_ref):
    s = stats_ref[...]
    mean = s[0:1, :] * (1.0 / _N_NODES)
    var = s[1:2, :] * (1.0 / _N_NODES) - mean * mean
    inv = lax.rsqrt(var + _EPS)
    out_ref[...] = jnp.tanh(
        (pre_ref[...] - mean) * (inv * gam_ref[...]) + bet_ref[...])


def _bn_call(pre, stats, gam, bet):
    return pl.pallas_call(
        _bn_body,
        grid=(_N_PAD // _NODE_BLK,),
        in_specs=[pl.BlockSpec((_NODE_BLK, _D), lambda i: (i, 0)),
                  pl.BlockSpec((2, _D), lambda i: (0, 0)),
                  pl.BlockSpec((1, _D), lambda i: (0, 0)),
                  pl.BlockSpec((1, _D), lambda i: (0, 0))],
        out_specs=pl.BlockSpec((_NODE_BLK, _D), lambda i: (i, 0)),
        out_shape=jax.ShapeDtypeStruct((_N_PAD, _D), jnp.float32),
    )(pre, stats, gam, bet)


def kernel(x, edge_index, edge_attr, norm, w_loop, w_in, loop_rel, w_bias,
           bn_gamma, bn_beta):
    src = edge_index[0].astype(jnp.int32)
    dst = edge_index[1].astype(jnp.int32)

    ax = jnp.asarray(_AX)
    ab = jnp.asarray(_AB)
    m3 = jnp.dot(jnp.asarray(_M2), w_in, precision=lax.Precision.HIGHEST)
    shift = loop_rel[0][jnp.asarray(_SHIFT_IDX)]
    wle = jnp.dot(shift, w_loop, precision=lax.Precision.HIGHEST)

    x_pad = jnp.pad(x, ((0, _N_PAD - _N_NODES), (0, 0)))
    norm_pad = jnp.pad(norm[:, 0], (0, _N_PAD - _N_NODES))

    fxt = _transform(ax, x_pad, 2048)
    fbt = _transform_pack(ab, edge_attr)

    sd = ((dst << 16) | src).astype(jnp.int32)
    aggt = _sc_edge_pass(fxt, fbt, sd, norm_pad)

    pre, stats = _nodes_call(aggt, m3, x_pad, wle, w_bias.reshape(1, _D))
    outp = _bn_call(pre, stats, bn_gamma.reshape(1, _D),
                    bn_beta.reshape(1, _D))
    return outp[:_N_NODES]
